# baseline (speedup 1.0000x reference)
import functools

import jax
import jax.numpy as jnp
from jax import lax
from jax.experimental import pallas as pl
from jax.experimental.pallas import tpu as pltpu

WS = 8
KNN = 8
C = 128
NB = 16
NN = NB * NB
P = WS * WS
FD = P * C
ROWS = NN * P
F32 = jnp.float32
BF16 = jnp.bfloat16
CCH = 2048


def _gelu(x):
    return 0.5 * x * (1.0 + lax.erf(x * 0.7071067811865476))


HI = lax.Precision.HIGHEST
MM = lax.Precision.DEFAULT


def _knn_body(x_ref, idx_ref, val_ref):
    X = x_ref[:]
    G = lax.dot_general(X, X, (((1,), (1,)), ((), ())),
                        preferred_element_type=F32)
    sq = jnp.sum(X * X, axis=1, keepdims=True)
    ones = jnp.ones((NN, 1), F32)
    BJ = lax.dot_general(ones, sq, (((1,), (1,)), ((), ())),
                         precision=HI, preferred_element_type=F32)
    D = jnp.sqrt(jnp.maximum((sq + BJ) - 2.0 * G, 1e-12))
    jcol = lax.broadcasted_iota(jnp.int32, (NN, NN), 1)
    idxs, vals = [], []
    for _ in range(KNN):
        m = jnp.min(D, axis=1, keepdims=True)
        cand = jnp.where(D == m, jcol, NN)
        am = jnp.min(cand, axis=1, keepdims=True)
        idxs.append(am)
        vals.append(m)
        D = jnp.where(jcol == am, jnp.float32(jnp.inf), D)
    idx_ref[:] = jnp.concatenate(idxs, axis=1)
    val_ref[:] = jnp.concatenate(vals, axis=1)


def _knn_call(Xn):
    return pl.pallas_call(
        _knn_body,
        out_shape=(
            jax.ShapeDtypeStruct((NN, KNN), jnp.int32),
            jax.ShapeDtypeStruct((NN, KNN), F32),
        ),
    )(Xn)


def _conv_body(fp_ref, fc_ref, fn_ref, w_ref, b_ref, o_ref, scratch, *,
               blk, cin, cout, w, check_y, pad, apply_gelu, has_bias, nblk,
               out_dtype):
    i = pl.program_id(0)
    scratch[pl.ds(pad, blk), :] = fc_ref[:].astype(F32)
    scratch[pl.ds(0, pad), :] = fp_ref[pl.ds(blk - pad, pad), :].astype(F32)
    scratch[pl.ds(pad + blk, pad), :] = fn_ref[pl.ds(0, pad), :].astype(F32)

    @pl.when(i == 0)
    def _():
        scratch[pl.ds(0, pad), :] = jnp.zeros((pad, cin), F32)

    @pl.when(i == nblk - 1)
    def _():
        scratch[pl.ds(pad + blk, pad), :] = jnp.zeros((pad, cin), F32)

    ri = lax.broadcasted_iota(jnp.int32, (blk, 1), 0)
    px = ri % w
    py = (ri // w) % w
    mxs = {dx: ((px + dx >= 0) & (px + dx < w)).astype(F32)
           for dx in (-1, 0, 1)}
    if check_y:
        mys = {dy: ((py + dy >= 0) & (py + dy < w)).astype(F32)
               for dy in (-1, 0, 1)}
    acc = jnp.zeros((blk, cout), F32)
    for dy in (-1, 0, 1):
        for dx in (-1, 0, 1):
            off = dy * w + dx
            m = mxs[dx]
            if check_y:
                m = m * mys[dy]
            sh = scratch[pl.ds(pad + off, blk), :]
            acc = acc + jnp.dot((sh * m).astype(BF16),
                                w_ref[3 * (dy + 1) + dx + 1],
                                precision=MM, preferred_element_type=F32)
    if has_bias:
        acc = acc + b_ref[:]
    if apply_gelu:
        acc = _gelu(acc)
    o_ref[:] = acc.astype(out_dtype)


def _conv_call(F, W9, bias, *, w, check_y, apply_gelu, out_dtype=BF16,
               blk=4096):
    rows, cin = F.shape
    cout = W9.shape[-1]
    nblk = rows // blk
    pad = w + 8
    body = functools.partial(
        _conv_body, blk=blk, cin=cin, cout=cout, w=w, check_y=check_y,
        pad=pad, apply_gelu=apply_gelu, has_bias=bias is not None, nblk=nblk,
        out_dtype=out_dtype)
    args = (F, F, F, W9) + ((bias,) if bias is not None else ())
    if bias is None:
        def body2(fp_ref, fc_ref, fn_ref, w_ref, o_ref, scratch):
            return body(fp_ref, fc_ref, fn_ref, w_ref, None, o_ref, scratch)
    else:
        body2 = body
    in_specs = [
        pl.BlockSpec((blk, cin), lambda i: (jnp.maximum(i - 1, 0), 0)),
        pl.BlockSpec((blk, cin), lambda i: (i, 0)),
        pl.BlockSpec((blk, cin), lambda i: (jnp.minimum(i + 1, nblk - 1), 0)),
        pl.BlockSpec((9, cin, cout), lambda i: (0, 0, 0)),
    ]
    if bias is not None:
        in_specs.append(pl.BlockSpec((1, cout), lambda i: (0, 0)))
    return pl.pallas_call(
        body2,
        grid=(nblk,),
        in_specs=in_specs,
        out_specs=pl.BlockSpec((blk, cout), lambda i: (i, 0)),
        out_shape=jax.ShapeDtypeStruct((rows, cout), out_dtype),
        scratch_shapes=[pltpu.VMEM((blk + 2 * pad, cin), F32)],
    )(*args)


def _dual_conv_body(fd_ref, fs_ref, wd_ref, ws_ref, b_ref, dc_ref, sp_ref,
                    *, blk):
    ri = lax.broadcasted_iota(jnp.int32, (blk, 1), 0)
    px = ri % WS
    py = (ri // WS) % WS
    mxs = {dx: ((px + dx >= 0) & (px + dx < WS)).astype(F32)
           for dx in (-1, 0, 1)}
    mys = {dy: ((py + dy >= 0) & (py + dy < WS)).astype(F32)
           for dy in (-1, 0, 1)}
    Fd = fd_ref[:].astype(F32)
    Fs = fs_ref[:].astype(F32)
    accd = jnp.zeros((blk, C), F32)
    accs = jnp.zeros((blk, C), F32)
    for dy in (-1, 0, 1):
        for dx in (-1, 0, 1):
            off = dy * WS + dx
            m = mxs[dx] * mys[dy]
            k = 3 * (dy + 1) + dx + 1
            if off > 0:
                pz = jnp.zeros((off, C), F32)
                shd = jnp.concatenate([Fd[off:, :], pz], axis=0)
                shs = jnp.concatenate([Fs[off:, :], pz], axis=0)
            elif off < 0:
                pz = jnp.zeros((-off, C), F32)
                shd = jnp.concatenate([pz, Fd[:off, :]], axis=0)
                shs = jnp.concatenate([pz, Fs[:off, :]], axis=0)
            else:
                shd, shs = Fd, Fs
            accd = accd + jnp.dot((shd * m).astype(BF16), wd_ref[k],
                                  precision=MM, preferred_element_type=F32)
            accs = accs + jnp.dot((shs * m).astype(BF16), ws_ref[k],
                                  precision=MM, preferred_element_type=F32)
    dc_ref[:] = _gelu(accd + b_ref[:]).astype(BF16)
    sp_ref[:] = accs.astype(BF16)


def _dual_conv_call(Fd, Fs, W9d, W9s, bd, blk=4096):
    nblk = ROWS // blk
    body = functools.partial(_dual_conv_body, blk=blk)
    return pl.pallas_call(
        body,
        grid=(nblk,),
        in_specs=[
            pl.BlockSpec((blk, C), lambda i: (i, 0)),
            pl.BlockSpec((blk, C), lambda i: (i, 0)),
            pl.BlockSpec((9, C, C), lambda i: (0, 0, 0)),
            pl.BlockSpec((9, C, C), lambda i: (0, 0, 0)),
            pl.BlockSpec((1, C), lambda i: (0, 0)),
        ],
        out_specs=(pl.BlockSpec((blk, C), lambda i: (i, 0)),
                   pl.BlockSpec((blk, C), lambda i: (i, 0))),
        out_shape=(jax.ShapeDtypeStruct((ROWS, C), BF16),
                   jax.ShapeDtypeStruct((ROWS, C), BF16)),
    )(Fd, Fs, W9d, W9s, bd)


def _graph_body(dn_ref, sp_ref, idx_ref, val_ref, w2_ref,
                ba_ref, bst_ref, snew_ref, dnew_ref):
    dist = val_ref[:]
    sigma = jnp.sum(dist, axis=1, keepdims=True) / jnp.float32(KNN)
    e = jnp.exp(-dist / (sigma ** 2))

    nch = FD // CCH

    def u_step(t, u):
        Dc = dn_ref[:, pl.ds(t * CCH, CCH)]
        Wc = w2_ref[:, pl.ds(t * CCH, CCH)]
        return u + lax.dot_general(Dc, Wc, (((1,), (1,)), ((), ())),
                                   precision=MM, preferred_element_type=F32)

    u = lax.fori_loop(0, nch, u_step, jnp.zeros((NN, 2), F32))
    u_src = u[:, 0:1]
    u_dst = u[:, 1:2]

    idxm = idx_ref[:]
    jcol = lax.broadcasted_iota(jnp.int32, (NN, NN), 1)
    Pks, a_cols = [], []
    for k in range(KNN):
        Pk = (jcol == idxm[:, k:k + 1]).astype(BF16)
        Pks.append(Pk)
        usk = jnp.dot(Pk.astype(F32), u_src, precision=MM,
                      preferred_element_type=F32)
        t = (usk + u_dst) * (1.0 / 16.0) + ba_ref[:]
        a_cols.append(jnp.where(t >= 0, t, 0.01 * t))
    a = jnp.concatenate(a_cols, axis=1)
    amax = jnp.max(a, axis=1, keepdims=True)
    ex = jnp.exp(a - amax)
    alpha = ex / jnp.sum(ex, axis=1, keepdims=True)

    A = jnp.zeros((NN, NN), F32)
    for k in range(KNN):
        A = A + alpha[:, k:k + 1] * Pks[k].astype(F32)
    A = A.astype(BF16)

    def col_step(t, _):
        ds = pl.ds(t * CCH, CCH)
        Dc = dn_ref[:, ds]
        dnew_ref[:, ds] = jnp.dot(A, Dc, precision=MM,
                                  preferred_element_type=F32).astype(BF16)
        Sc = sp_ref[:, ds]
        bst = bst_ref[:, ds]
        zmin = zmax = None
        for k in range(KNN):
            g = jnp.dot(Pks[k], Sc, precision=MM, preferred_element_type=F32)
            z = e[:, k:k + 1] * g + bst
            zmin = z if zmin is None else jnp.minimum(zmin, z)
            zmax = z if zmax is None else jnp.maximum(zmax, z)
        snew_ref[:, ds] = jnp.maximum(_gelu(zmin), _gelu(zmax)).astype(BF16)
        return 0

    lax.fori_loop(0, nch, col_step, 0)


def _graph_call(Dn, Sp, idx, vals, w2, ba, bst):
    return pl.pallas_call(
        _graph_body,
        out_shape=(
            jax.ShapeDtypeStruct((NN, FD), BF16),
            jax.ShapeDtypeStruct((NN, FD), BF16),
        ),
    )(Dn, Sp, idx, vals, w2, ba, bst)


def _w9(W):
    cout, cin = W.shape[0], W.shape[1]
    return jnp.transpose(W, (2, 3, 1, 0)).reshape(9, cin, cout).astype(BF16)


def _w2(Wa):
    arr = jnp.tile(Wa[0], (1, WS // 2, WS // 2))
    arr = jnp.transpose(arr, (1, 2, 0)).reshape(P, 2 * C)
    wsrc = arr[:, :C].reshape(FD)
    wdst = arr[:, C:].reshape(FD)
    return jnp.stack([wsrc, wdst], axis=0).astype(BF16)


def _patchify(x):
    t = x[0].reshape(C, NB, WS, NB, WS)
    return jnp.transpose(t, (1, 3, 2, 4, 0)).reshape(ROWS, C)


def _unpatch(F):
    t = F.reshape(NB, NB, WS, WS, C)
    return jnp.transpose(t, (0, 2, 1, 3, 4)).reshape(NB * WS * NB * WS, C)


def kernel(x, Ws0, bs0, Wd0, bd0, Wa0, ba0, Ws1, bs1, Wd1, bd1, Wa1, ba1,
           Wo, bo):
    F0 = _patchify(x)
    idx, vals = _knn_call(F0.reshape(NN, FD))

    layers = ((Ws0, bs0, Wd0, bd0, Wa0, ba0), (Ws1, bs1, Wd1, bd1, Wa1, ba1))
    Fs = Fd = F0.astype(BF16)
    for Ws_, bs_, Wd_, bd_, Wa_, ba_ in layers:
        Dc, Sp = _dual_conv_call(Fd, Fs, _w9(Wd_), _w9(Ws_),
                                 bd_.reshape(1, C))
        s_n, d_n = _graph_call(
            Dc.reshape(NN, FD), Sp.reshape(NN, FD), idx, vals,
            _w2(Wa_), ba_.reshape(1, 1),
            jnp.tile(bs_, P).reshape(1, FD))
        Fs = s_n.reshape(ROWS, C)
        Fd = d_n.reshape(ROWS, C)

    Xi = jnp.concatenate([_unpatch(Fs), _unpatch(Fd)], axis=1)
    M = _conv_call(Xi, _w9(Wo), bo.reshape(1, C), w=NB * WS, check_y=False,
                   apply_gelu=True, out_dtype=F32)
    H = NB * WS
    return jnp.transpose(M.reshape(H, H, C), (2, 0, 1))[None]


# speedup vs baseline: 1.0182x; 1.0182x over previous
import functools

import jax
import jax.numpy as jnp
from jax import lax
from jax.experimental import pallas as pl
from jax.experimental.pallas import tpu as pltpu

WS = 8
KNN = 8
C = 128
NB = 16
NN = NB * NB
P = WS * WS
FD = P * C
ROWS = NN * P
F32 = jnp.float32
BF16 = jnp.bfloat16
CCH = 2048


def _gelu(x):
    return 0.5 * x * (1.0 + lax.erf(x * 0.7071067811865476))


HI = lax.Precision.HIGHEST
MM = lax.Precision.DEFAULT


def _knn_body(x_ref, idx_ref, val_ref):
    X = x_ref[:]
    G = lax.dot_general(X, X, (((1,), (1,)), ((), ())),
                        preferred_element_type=F32)
    sq = jnp.sum(X * X, axis=1, keepdims=True)
    ones = jnp.ones((NN, 1), F32)
    BJ = lax.dot_general(ones, sq, (((1,), (1,)), ((), ())),
                         precision=HI, preferred_element_type=F32)
    D = jnp.sqrt(jnp.maximum((sq + BJ) - 2.0 * G, 1e-12))
    jcol = lax.broadcasted_iota(jnp.int32, (NN, NN), 1)
    idxs, vals = [], []
    for _ in range(KNN):
        m = jnp.min(D, axis=1, keepdims=True)
        cand = jnp.where(D == m, jcol, NN)
        am = jnp.min(cand, axis=1, keepdims=True)
        idxs.append(am)
        vals.append(m)
        D = jnp.where(jcol == am, jnp.float32(jnp.inf), D)
    idx_ref[:] = jnp.concatenate(idxs, axis=1)
    val_ref[:] = jnp.concatenate(vals, axis=1)


def _knn_call(Xn):
    return pl.pallas_call(
        _knn_body,
        out_shape=(
            jax.ShapeDtypeStruct((NN, KNN), jnp.int32),
            jax.ShapeDtypeStruct((NN, KNN), F32),
        ),
    )(Xn)


def _conv_body(fp_ref, fc_ref, fn_ref, w_ref, b_ref, o_ref, scratch, *,
               blk, cin, cout, w, check_y, pad, apply_gelu, has_bias, nblk,
               out_dtype):
    i = pl.program_id(0)
    scratch[pl.ds(pad, blk), :] = fc_ref[:].astype(F32)
    scratch[pl.ds(0, pad), :] = fp_ref[pl.ds(blk - pad, pad), :].astype(F32)
    scratch[pl.ds(pad + blk, pad), :] = fn_ref[pl.ds(0, pad), :].astype(F32)

    @pl.when(i == 0)
    def _():
        scratch[pl.ds(0, pad), :] = jnp.zeros((pad, cin), F32)

    @pl.when(i == nblk - 1)
    def _():
        scratch[pl.ds(pad + blk, pad), :] = jnp.zeros((pad, cin), F32)

    ri = lax.broadcasted_iota(jnp.int32, (blk, 1), 0)
    px = ri % w
    py = (ri // w) % w
    mxs = {dx: ((px + dx >= 0) & (px + dx < w)).astype(F32)
           for dx in (-1, 0, 1)}
    if check_y:
        mys = {dy: ((py + dy >= 0) & (py + dy < w)).astype(F32)
               for dy in (-1, 0, 1)}
    acc = jnp.zeros((blk, cout), F32)
    for dy in (-1, 0, 1):
        for dx in (-1, 0, 1):
            off = dy * w + dx
            m = mxs[dx]
            if check_y:
                m = m * mys[dy]
            sh = scratch[pl.ds(pad + off, blk), :]
            acc = acc + jnp.dot(sh * m, w_ref[3 * (dy + 1) + dx + 1],
                                precision=MM, preferred_element_type=F32)
    if has_bias:
        acc = acc + b_ref[:]
    if apply_gelu:
        acc = _gelu(acc)
    o_ref[:] = acc.astype(out_dtype)


def _conv_call(F, W9, bias, *, w, check_y, apply_gelu, out_dtype=BF16,
               blk=4096):
    rows, cin = F.shape
    cout = W9.shape[-1]
    nblk = rows // blk
    pad = w + 8
    body = functools.partial(
        _conv_body, blk=blk, cin=cin, cout=cout, w=w, check_y=check_y,
        pad=pad, apply_gelu=apply_gelu, has_bias=bias is not None, nblk=nblk,
        out_dtype=out_dtype)
    args = (F, F, F, W9) + ((bias,) if bias is not None else ())
    if bias is None:
        def body2(fp_ref, fc_ref, fn_ref, w_ref, o_ref, scratch):
            return body(fp_ref, fc_ref, fn_ref, w_ref, None, o_ref, scratch)
    else:
        body2 = body
    in_specs = [
        pl.BlockSpec((blk, cin), lambda i: (jnp.maximum(i - 1, 0), 0)),
        pl.BlockSpec((blk, cin), lambda i: (i, 0)),
        pl.BlockSpec((blk, cin), lambda i: (jnp.minimum(i + 1, nblk - 1), 0)),
        pl.BlockSpec((9, cin, cout), lambda i: (0, 0, 0)),
    ]
    if bias is not None:
        in_specs.append(pl.BlockSpec((1, cout), lambda i: (0, 0)))
    return pl.pallas_call(
        body2,
        grid=(nblk,),
        in_specs=in_specs,
        out_specs=pl.BlockSpec((blk, cout), lambda i: (i, 0)),
        out_shape=jax.ShapeDtypeStruct((rows, cout), out_dtype),
        scratch_shapes=[pltpu.VMEM((blk + 2 * pad, cin), F32)],
    )(*args)


def _dual_conv_body(fd_ref, fs_ref, wd_ref, ws_ref, b_ref, dc_ref, sp_ref,
                    *, blk):
    ri = lax.broadcasted_iota(jnp.int32, (blk, 1), 0)
    px = ri % WS
    py = (ri // WS) % WS
    mxs = {dx: ((px + dx >= 0) & (px + dx < WS)).astype(F32)
           for dx in (-1, 0, 1)}
    mys = {dy: ((py + dy >= 0) & (py + dy < WS)).astype(F32)
           for dy in (-1, 0, 1)}
    Fd = fd_ref[:].astype(F32)
    Fs = fs_ref[:].astype(F32)
    accd = jnp.zeros((blk, C), F32)
    accs = jnp.zeros((blk, C), F32)
    for dy in (-1, 0, 1):
        for dx in (-1, 0, 1):
            off = dy * WS + dx
            m = mxs[dx] * mys[dy]
            k = 3 * (dy + 1) + dx + 1
            if off > 0:
                pz = jnp.zeros((off, C), F32)
                shd = jnp.concatenate([Fd[off:, :], pz], axis=0)
                shs = jnp.concatenate([Fs[off:, :], pz], axis=0)
            elif off < 0:
                pz = jnp.zeros((-off, C), F32)
                shd = jnp.concatenate([pz, Fd[:off, :]], axis=0)
                shs = jnp.concatenate([pz, Fs[:off, :]], axis=0)
            else:
                shd, shs = Fd, Fs
            accd = accd + jnp.dot(shd * m, wd_ref[k],
                                  precision=MM, preferred_element_type=F32)
            accs = accs + jnp.dot(shs * m, ws_ref[k],
                                  precision=MM, preferred_element_type=F32)
    dc_ref[:] = _gelu(accd + b_ref[:]).astype(BF16)
    sp_ref[:] = accs.astype(BF16)


def _dual_conv_call(Fd, Fs, W9d, W9s, bd, blk=4096):
    nblk = ROWS // blk
    body = functools.partial(_dual_conv_body, blk=blk)
    return pl.pallas_call(
        body,
        grid=(nblk,),
        in_specs=[
            pl.BlockSpec((blk, C), lambda i: (i, 0)),
            pl.BlockSpec((blk, C), lambda i: (i, 0)),
            pl.BlockSpec((9, C, C), lambda i: (0, 0, 0)),
            pl.BlockSpec((9, C, C), lambda i: (0, 0, 0)),
            pl.BlockSpec((1, C), lambda i: (0, 0)),
        ],
        out_specs=(pl.BlockSpec((blk, C), lambda i: (i, 0)),
                   pl.BlockSpec((blk, C), lambda i: (i, 0))),
        out_shape=(jax.ShapeDtypeStruct((ROWS, C), BF16),
                   jax.ShapeDtypeStruct((ROWS, C), BF16)),
    )(Fd, Fs, W9d, W9s, bd)


def _graph_body(dn_ref, sp_ref, idx_ref, val_ref, w2_ref,
                ba_ref, bst_ref, snew_ref, dnew_ref):
    dist = val_ref[:]
    sigma = jnp.sum(dist, axis=1, keepdims=True) / jnp.float32(KNN)
    e = jnp.exp(-dist / (sigma ** 2))

    nch = FD // CCH

    def u_step(t, u):
        Dc = dn_ref[:, pl.ds(t * CCH, CCH)]
        Wc = w2_ref[:, pl.ds(t * CCH, CCH)]
        return u + lax.dot_general(Dc, Wc, (((1,), (1,)), ((), ())),
                                   precision=MM, preferred_element_type=F32)

    u = lax.fori_loop(0, nch, u_step, jnp.zeros((NN, 2), F32))
    u_src = u[:, 0:1]
    u_dst = u[:, 1:2]

    idxm = idx_ref[:]
    jcol = lax.broadcasted_iota(jnp.int32, (NN, NN), 1)
    Pks, a_cols = [], []
    for k in range(KNN):
        Pk = (jcol == idxm[:, k:k + 1]).astype(BF16)
        Pks.append(Pk)
        usk = jnp.dot(Pk.astype(F32), u_src, precision=MM,
                      preferred_element_type=F32)
        t = (usk + u_dst) * (1.0 / 16.0) + ba_ref[:]
        a_cols.append(jnp.where(t >= 0, t, 0.01 * t))
    a = jnp.concatenate(a_cols, axis=1)
    amax = jnp.max(a, axis=1, keepdims=True)
    ex = jnp.exp(a - amax)
    alpha = ex / jnp.sum(ex, axis=1, keepdims=True)

    A = jnp.zeros((NN, NN), F32)
    for k in range(KNN):
        A = A + alpha[:, k:k + 1] * Pks[k].astype(F32)
    A = A.astype(BF16)

    def col_step(t, _):
        ds = pl.ds(t * CCH, CCH)
        Dc = dn_ref[:, ds]
        dnew_ref[:, ds] = jnp.dot(A, Dc, precision=MM,
                                  preferred_element_type=F32).astype(BF16)
        Sc = sp_ref[:, ds]
        bst = bst_ref[:, ds]
        zmin = zmax = None
        for k in range(KNN):
            g = jnp.dot(Pks[k], Sc, precision=MM, preferred_element_type=F32)
            z = e[:, k:k + 1] * g + bst
            zmin = z if zmin is None else jnp.minimum(zmin, z)
            zmax = z if zmax is None else jnp.maximum(zmax, z)
        snew_ref[:, ds] = jnp.maximum(_gelu(zmin), _gelu(zmax)).astype(BF16)
        return 0

    lax.fori_loop(0, nch, col_step, 0)


def _graph_call(Dn, Sp, idx, vals, w2, ba, bst):
    return pl.pallas_call(
        _graph_body,
        out_shape=(
            jax.ShapeDtypeStruct((NN, FD), BF16),
            jax.ShapeDtypeStruct((NN, FD), BF16),
        ),
    )(Dn, Sp, idx, vals, w2, ba, bst)


def _w9(W):
    cout, cin = W.shape[0], W.shape[1]
    return jnp.transpose(W, (2, 3, 1, 0)).reshape(9, cin, cout).astype(BF16)


def _w2(Wa):
    arr = jnp.tile(Wa[0], (1, WS // 2, WS // 2))
    arr = jnp.transpose(arr, (1, 2, 0)).reshape(P, 2 * C)
    wsrc = arr[:, :C].reshape(FD)
    wdst = arr[:, C:].reshape(FD)
    return jnp.stack([wsrc, wdst], axis=0).astype(BF16)


def _patchify(x):
    t = x[0].reshape(C, NB, WS, NB, WS)
    return jnp.transpose(t, (1, 3, 2, 4, 0)).reshape(ROWS, C)


def _unpatch(F):
    t = F.reshape(NB, NB, WS, WS, C)
    return jnp.transpose(t, (0, 2, 1, 3, 4)).reshape(NB * WS * NB * WS, C)


def kernel(x, Ws0, bs0, Wd0, bd0, Wa0, ba0, Ws1, bs1, Wd1, bd1, Wa1, ba1,
           Wo, bo):
    F0 = _patchify(x)
    idx, vals = _knn_call(F0.reshape(NN, FD))

    layers = ((Ws0, bs0, Wd0, bd0, Wa0, ba0), (Ws1, bs1, Wd1, bd1, Wa1, ba1))
    Fs = Fd = F0.astype(BF16)
    for Ws_, bs_, Wd_, bd_, Wa_, ba_ in layers:
        Dc, Sp = _dual_conv_call(Fd, Fs, _w9(Wd_), _w9(Ws_),
                                 bd_.reshape(1, C))
        s_n, d_n = _graph_call(
            Dc.reshape(NN, FD), Sp.reshape(NN, FD), idx, vals,
            _w2(Wa_), ba_.reshape(1, 1),
            jnp.tile(bs_, P).reshape(1, FD))
        Fs = s_n.reshape(ROWS, C)
        Fd = d_n.reshape(ROWS, C)

    Xi = jnp.concatenate([_unpatch(Fs), _unpatch(Fd)], axis=1)
    M = _conv_call(Xi, _w9(Wo), bo.reshape(1, C), w=NB * WS, check_y=False,
                   apply_gelu=True, out_dtype=F32)
    H = NB * WS
    return jnp.transpose(M.reshape(H, H, C), (2, 0, 1))[None]
